# baseline (speedup 1.0000x reference)
import functools

import jax
import jax.numpy as jnp
from jax import lax
from jax.experimental import pallas as pl
from jax.experimental.pallas import tpu as pltpu
from jax.experimental.pallas import tpu_sc as plsc

_VOCAB = 1000000
_D = 64
_B = 4096
_S = 200

_NC, _NS = 2, 16
_NW = _NC * _NS
_ROWS = _B * _S
_ROWS_W = _ROWS // _NW
_G = 100
_NG = _ROWS_W // _G
_SEQ_W = _ROWS_W // _S


def _pos_encoding():
    pos = jnp.arange(_S, dtype=jnp.float32)[:, None]
    i = jnp.arange(_D)[None, :]
    angle_rates = 1.0 / jnp.power(10000.0, (2.0 * (i // 2)) / jnp.float32(_D))
    angle_rads = pos * angle_rates
    return jnp.where(i % 2 == 0, jnp.sin(angle_rads), jnp.cos(angle_rads)).astype(
        jnp.float32
    )


@functools.partial(
    pl.kernel,
    out_type=jax.ShapeDtypeStruct((_ROWS, _D), jnp.float32),
    mesh=plsc.VectorSubcoreMesh(core_axis_name="c", subcore_axis_name="s"),
    scratch_types=[
        pltpu.VMEM((_NG, _G), jnp.int32),
        pltpu.VMEM((_S, _D), jnp.float32),
        pltpu.VMEM((_S, _D), jnp.float32),
        pltpu.SemaphoreType.DMA,
    ],
    compiler_params=pltpu.CompilerParams(use_tc_tiling_on_sc=False),
)
def _embed(x_hbm, table_hbm, pe_hbm, out_hbm, idx_v, pe_v, rows_v, sem):
    wid = lax.axis_index("s") * _NC + lax.axis_index("c")
    pltpu.sync_copy(x_hbm.at[wid], idx_v)
    pltpu.sync_copy(pe_hbm, pe_v)
    base = wid * _ROWS_W

    def seq_body(s, carry):
        j = s * 2
        cp0 = pltpu.async_copy(
            table_hbm.at[idx_v.at[j]], rows_v.at[pl.ds(0, _G)], sem
        )
        cp1 = pltpu.async_copy(
            table_hbm.at[idx_v.at[j + 1]], rows_v.at[pl.ds(_G, _G)], sem
        )
        cp0.wait()
        cp1.wait()

        def row_body(r, c2):
            for c in range(_D // 16):
                sl = pl.ds(c * 16, 16)
                rows_v[r, sl] = rows_v[r, sl] + pe_v[r, sl]
            return c2

        lax.fori_loop(0, _S, row_body, 0)
        pltpu.sync_copy(rows_v, out_hbm.at[pl.ds(base + s * _S, _S)])
        return carry

    lax.fori_loop(0, _SEQ_W, seq_body, 0)


def kernel(x, table):
    pe = _pos_encoding()
    x3 = x.reshape(_NW, _NG, _G)
    out = _embed(x3, table, pe)
    return out.reshape(_B, _S, _D)


# speedup vs baseline: 1.1469x; 1.1469x over previous
import functools

import jax
import jax.numpy as jnp
from jax import lax
from jax.experimental import pallas as pl
from jax.experimental.pallas import tpu as pltpu
from jax.experimental.pallas import tpu_sc as plsc

_VOCAB = 1000000
_D = 64
_B = 4096
_S = 200

_NC, _NS = 2, 16
_NW = _NC * _NS
_SEQ_W = _B // _NW
_G0 = 96
_G1 = _S - _G0


def _pos_encoding():
    pos = jnp.arange(_S, dtype=jnp.float32)[:, None]
    i = jnp.arange(_D)[None, :]
    angle_rates = 1.0 / jnp.power(10000.0, (2.0 * (i // 2)) / jnp.float32(_D))
    angle_rads = pos * angle_rates
    return jnp.where(i % 2 == 0, jnp.sin(angle_rads), jnp.cos(angle_rads)).astype(
        jnp.float32
    )


@functools.partial(
    pl.kernel,
    out_type=jax.ShapeDtypeStruct((_B, _S, _D), jnp.float32),
    mesh=plsc.VectorSubcoreMesh(core_axis_name="c", subcore_axis_name="s"),
    scratch_types=[
        pltpu.VMEM((_SEQ_W, _S), jnp.int32),
        pltpu.VMEM((_S, _D), jnp.float32),
        pltpu.VMEM((_S, _D), jnp.float32),
        pltpu.VMEM((_S, _D), jnp.float32),
        pltpu.VMEM((_S, _D), jnp.float32),
        pltpu.VMEM((_S, _D), jnp.float32),
        pltpu.SemaphoreType.DMA,
        pltpu.SemaphoreType.DMA,
        pltpu.SemaphoreType.DMA,
        pltpu.SemaphoreType.DMA,
    ],
    compiler_params=pltpu.CompilerParams(use_tc_tiling_on_sc=False),
)
def _embed(
    x_hbm, table_hbm, pe_hbm, out_hbm,
    idx_v, pe_v, bg0, bg1, bo0, bo1, sg0, sg1, so0, so1,
):
    wid = lax.axis_index("s") * _NC + lax.axis_index("c")
    seq0 = wid * _SEQ_W
    pltpu.sync_copy(x_hbm.at[pl.ds(seq0, _SEQ_W)], idx_v)
    pltpu.sync_copy(pe_hbm, pe_v)

    def start_gather(s, bg, sg):
        pltpu.async_copy(
            table_hbm.at[idx_v.at[s, pl.ds(0, _G0)]], bg.at[pl.ds(0, _G0)], sg
        )
        pltpu.async_copy(
            table_hbm.at[idx_v.at[s, pl.ds(_G0, _G1)]], bg.at[pl.ds(_G0, _G1)], sg
        )

    def wait_gather(bg, sg):
        pltpu.make_async_copy(table_hbm.at[pl.ds(0, _S)], bg, sg).wait()

    start_gather(0, bg0, sg0)
    start_gather(1, bg1, sg1)

    bufs = ((bg0, bo0, sg0, so0), (bg1, bo1, sg1, so1))

    def outer(i, carry):
        for b, (bg, bo, sg, so) in enumerate(bufs):
            s = i * 2 + b
            wait_gather(bg, sg)

            @pl.when(s >= 2)
            def _():
                pltpu.make_async_copy(bo, out_hbm.at[seq0 + s - 2], so).wait()

            def row_body(r, c2):
                for u in range(4):
                    rr = r * 4 + u
                    for c in range(_D // 16):
                        sl = pl.ds(c * 16, 16)
                        bo[rr, sl] = bg[rr, sl] + pe_v[rr, sl]
                return c2

            lax.fori_loop(0, _S // 4, row_body, 0)

            @pl.when(s + 2 < _SEQ_W)
            def _():
                start_gather(s + 2, bg, sg)

            pltpu.async_copy(bo, out_hbm.at[seq0 + s], so)
        return carry

    lax.fori_loop(0, _SEQ_W // 2, outer, 0)
    pltpu.make_async_copy(bo0, out_hbm.at[seq0 + _SEQ_W - 2], so0).wait()
    pltpu.make_async_copy(bo1, out_hbm.at[seq0 + _SEQ_W - 1], so1).wait()


def kernel(x, table):
    pe = _pos_encoding()
    return _embed(x, table, pe)


# speedup vs baseline: 1.1487x; 1.0016x over previous
import functools

import jax
import jax.numpy as jnp
from jax import lax
from jax.experimental import pallas as pl
from jax.experimental.pallas import tpu as pltpu
from jax.experimental.pallas import tpu_sc as plsc

_VOCAB = 1000000
_D = 64
_B = 4096
_S = 200

_NC, _NS = 2, 16
_NW = _NC * _NS
_SEQ_W = _B // _NW
_G0 = 96
_G1 = _S - _G0
_ROWS_O = _S * _D // 128


def _pos_encoding():
    pos = jnp.arange(_S, dtype=jnp.float32)[:, None]
    i = jnp.arange(_D)[None, :]
    angle_rates = 1.0 / jnp.power(10000.0, (2.0 * (i // 2)) / jnp.float32(_D))
    angle_rads = pos * angle_rates
    return jnp.where(i % 2 == 0, jnp.sin(angle_rads), jnp.cos(angle_rads)).astype(
        jnp.float32
    )


@functools.partial(
    pl.kernel,
    out_type=jax.ShapeDtypeStruct((_B * _S * _D // 128, 128), jnp.float32),
    mesh=plsc.VectorSubcoreMesh(core_axis_name="c", subcore_axis_name="s"),
    scratch_types=[
        pltpu.VMEM((_SEQ_W, _S), jnp.int32),
        pltpu.VMEM((_S * _D // 128, 128), jnp.float32),
        pltpu.VMEM((_S, _D), jnp.float32),
        pltpu.VMEM((_S, _D), jnp.float32),
        pltpu.VMEM((_S * _D // 128, 128), jnp.float32),
        pltpu.VMEM((_S * _D // 128, 128), jnp.float32),
        pltpu.SemaphoreType.DMA,
        pltpu.SemaphoreType.DMA,
        pltpu.SemaphoreType.DMA,
        pltpu.SemaphoreType.DMA,
    ],
    compiler_params=pltpu.CompilerParams(use_tc_tiling_on_sc=False),
)
def _embed(
    x_hbm, table_hbm, pe_hbm, out_hbm,
    idx_v, pe_v, bg0, bg1, bo0, bo1, sg0, sg1, so0, so1,
):
    wid = lax.axis_index("s") * _NC + lax.axis_index("c")
    seq0 = wid * _SEQ_W
    pltpu.sync_copy(x_hbm.at[pl.ds(seq0, _SEQ_W)], idx_v)
    pltpu.sync_copy(pe_hbm, pe_v)

    def start_gather(s, bg, sg):
        pltpu.async_copy(
            table_hbm.at[idx_v.at[s, pl.ds(0, _G0)]], bg.at[pl.ds(0, _G0)], sg
        )
        pltpu.async_copy(
            table_hbm.at[idx_v.at[s, pl.ds(_G0, _G1)]], bg.at[pl.ds(_G0, _G1)], sg
        )

    def wait_gather(bg, sg):
        pltpu.make_async_copy(table_hbm.at[pl.ds(0, _S)], bg, sg).wait()

    start_gather(0, bg0, sg0)
    start_gather(1, bg1, sg1)

    bufs = ((bg0, bo0, sg0, so0), (bg1, bo1, sg1, so1))

    def outer(i, carry):
        for b, (bg, bo, sg, so) in enumerate(bufs):
            s = i * 2 + b
            wait_gather(bg, sg)

            @pl.when(s >= 2)
            def _():
                pltpu.make_async_copy(
                    bo, out_hbm.at[pl.ds((seq0 + s - 2) * _ROWS_O, _ROWS_O)], so
                ).wait()

            def row_body(r, c2):
                for u in range(2):
                    p = r * 2 + u
                    for c in range(2 * _D // 16):
                        sl = pl.ds(c * 16, 16)
                        src_r = p * 2 + (c * 16) // _D
                        src_sl = pl.ds((c * 16) % _D, 16)
                        bo[p, sl] = bg[src_r, src_sl] + pe_v[p, sl]
                return c2

            lax.fori_loop(0, _ROWS_O // 2, row_body, 0)

            @pl.when(s + 2 < _SEQ_W)
            def _():
                start_gather(s + 2, bg, sg)

            pltpu.async_copy(
                bo, out_hbm.at[pl.ds((seq0 + s) * _ROWS_O, _ROWS_O)], so
            )
        return carry

    lax.fori_loop(0, _SEQ_W // 2, outer, 0)
    pltpu.make_async_copy(
        bo0, out_hbm.at[pl.ds((seq0 + _SEQ_W - 2) * _ROWS_O, _ROWS_O)], so0
    ).wait()
    pltpu.make_async_copy(
        bo1, out_hbm.at[pl.ds((seq0 + _SEQ_W - 1) * _ROWS_O, _ROWS_O)], so1
    ).wait()


def kernel(x, table):
    pe = _pos_encoding().reshape(_ROWS_O, 128)
    return _embed(x, table, pe).reshape(_B, _S, _D)
